# baseline (speedup 1.0000x reference)
import functools

import jax
import jax.numpy as jnp
from jax.experimental import pallas as pl
from jax.experimental.pallas import tpu as pltpu

B, T = 4096, 50
D_IN, ENC, HID = 11, 128, 128
STEPS = 50
BB = 512
HB = BB // 2
GX_CHUNK = 5
T_SEG = 25
D_INA = D_IN + 1


def _fused_kernel(x_ref, w1a_ref, w2a_ref, wih_ref, whha_ref,
                  cw1_ref, cb1_ref, cw2_ref, cb2_ref,
                  lw1_ref, lb1_ref, lw2_ref, lb2_ref,
                  coords_ref, labels_ref, gx_scr):
    f32 = jnp.float32
    bf16 = jnp.bfloat16
    w1a = w1a_ref[...].astype(bf16)
    w2a = w2a_ref[...].astype(bf16)
    wih = wih_ref[...].astype(bf16)
    whha = whha_ref[...].astype(bf16)
    ones_chunk = jnp.ones((GX_CHUNK * BB, 1), dtype=bf16)
    ones_h = jnp.ones((HB, 1), dtype=bf16)

    def sig(x):
        return jnp.bfloat16(0.5) * jnp.tanh(jnp.bfloat16(0.5) * x) + jnp.bfloat16(0.5)

    def act(gates, c):
        i_t = sig(gates[:, 0 * HID:1 * HID])
        f_t = sig(gates[:, 1 * HID:2 * HID])
        g_t = jnp.tanh(gates[:, 2 * HID:3 * HID])
        o_t = sig(gates[:, 3 * HID:4 * HID])
        c_new = f_t * c + i_t * g_t
        h_new = o_t * jnp.tanh(c_new)
        return h_new, c_new

    zero = jnp.zeros((HB, HID), dtype=bf16)
    hs = [zero, zero]
    cs = [zero, zero]
    for seg in range(T // T_SEG):
        for c in range(T_SEG // GX_CHUNK):
            ct = seg * T_SEG + c * GX_CHUNK
            xc = x_ref[ct:ct + GX_CHUNK].reshape(GX_CHUNK * BB, D_INA)
            z1 = jnp.maximum(jnp.dot(xc, w1a, preferred_element_type=f32).astype(bf16), 0.0)
            z1 = jnp.concatenate([z1, ones_chunk], axis=1)
            z2 = jnp.maximum(jnp.dot(z1, w2a, preferred_element_type=f32).astype(bf16), 0.0)
            gx = jnp.dot(z2, wih, preferred_element_type=f32).astype(bf16)
            gx_scr[c * GX_CHUNK:(c + 1) * GX_CHUNK] = gx.reshape(GX_CHUNK, BB, 4 * HID)
        for t in range(T_SEG):
            gx = gx_scr[t]
            for q in range(2):
                hq = jnp.concatenate([hs[q], ones_h], axis=1)
                gq = gx[q * HB:(q + 1) * HB] + jnp.dot(
                    hq, whha, preferred_element_type=f32).astype(bf16)
                hs[q], cs[q] = act(gq, cs[q])
    h = jnp.concatenate(hs, axis=0).astype(f32)

    hc = jnp.maximum(jnp.dot(h, cw1_ref[...], preferred_element_type=f32)
                     + cb1_ref[...], 0.0)
    coords_ref[...] = jnp.dot(hc, cw2_ref[...], preferred_element_type=f32) + cb2_ref[...]
    hl = jnp.maximum(jnp.dot(h, lw1_ref[...], preferred_element_type=f32)
                     + lb1_ref[...], 0.0)
    labels_ref[...] = jnp.dot(hl, lw2_ref[...], preferred_element_type=f32) + lb2_ref[...]


def _full(shape):
    return pl.BlockSpec(shape, lambda i: (0,) * len(shape))


@functools.partial(jax.jit, static_argnames=("interpret",))
def _run(x, W1a, W2a, W_ih, Whha,
         coord_W1, coord_b1, coord_W2, coord_b2,
         lab_W1, lab_b1, lab_W2, lab_b2, interpret=False):
    n_tiles = B // BB
    out_shapes = (
        jax.ShapeDtypeStruct((B, 3 * STEPS), jnp.float32),
        jax.ShapeDtypeStruct((B, STEPS), jnp.float32),
    )
    return pl.pallas_call(
        _fused_kernel,
        grid=(n_tiles,),
        in_specs=[
            pl.BlockSpec((T, BB, D_INA), lambda i: (0, i, 0)),
            _full((D_INA, ENC)),
            _full((ENC + 1, ENC)),
            _full((ENC, 4 * HID)),
            _full((HID + 1, 4 * HID)),
            _full((HID, HID)), _full((1, HID)),
            _full((HID, 3 * STEPS)), _full((1, 3 * STEPS)),
            _full((HID, HID // 2)), _full((1, HID // 2)),
            _full((HID // 2, STEPS)), _full((1, STEPS)),
        ],
        out_specs=(
            pl.BlockSpec((BB, 3 * STEPS), lambda i: (i, 0)),
            pl.BlockSpec((BB, STEPS), lambda i: (i, 0)),
        ),
        out_shape=out_shapes,
        scratch_shapes=[pltpu.VMEM((T_SEG, BB, 4 * HID), jnp.bfloat16)],
        compiler_params=pltpu.CompilerParams(
            dimension_semantics=("parallel",),
        ),
        interpret=interpret,
    )(x, W1a, W2a, W_ih, Whha,
      coord_W1, coord_b1, coord_W2, coord_b2,
      lab_W1, lab_b1, lab_W2, lab_b2)


def kernel(obs_l, obs_c, obs_m, enc_W1, enc_b1, enc_W2, enc_b2,
           W_ih, W_hh, b_ih, b_hh,
           coord_W1, coord_b1, coord_W2, coord_b2,
           lab_W1, lab_b1, lab_W2, lab_b2):
    ones = jnp.ones((B, T, 1), dtype=obs_l.dtype)
    x = jnp.concatenate([obs_l, obs_c, obs_m, ones], axis=-1)
    x = jnp.swapaxes(x, 0, 1).astype(jnp.bfloat16)
    W1a = jnp.concatenate([enc_W1, enc_b1[None, :]], axis=0)
    W2a = jnp.concatenate([enc_W2, enc_b2[None, :]], axis=0)
    Whha = jnp.concatenate([W_hh, (b_ih + b_hh)[None, :]], axis=0)
    return _run(x, W1a, W2a, W_ih, Whha,
                coord_W1, coord_b1.reshape(1, HID), coord_W2, coord_b2.reshape(1, 3 * STEPS),
                lab_W1, lab_b1.reshape(1, HID // 2), lab_W2, lab_b2.reshape(1, STEPS))


# speedup vs baseline: 1.2068x; 1.2068x over previous
import functools

import jax
import jax.numpy as jnp
from jax.experimental import pallas as pl
from jax.experimental.pallas import tpu as pltpu

B, T = 4096, 50
D_IN, ENC, HID = 11, 128, 128
STEPS = 50
BB = 512
HB = BB // 2
GX_CHUNK = 5
T_SEG = 25
D_INA = D_IN + 1


def _fused_kernel(x_ref, w1a_ref, w2a_ref, wih_ref, whha_ref,
                  cw1_ref, cb1_ref, cw2_ref, cb2_ref,
                  lw1_ref, lb1_ref, lw2_ref, lb2_ref,
                  coords_ref, labels_ref, gx_scr):
    f32 = jnp.float32
    bf16 = jnp.bfloat16
    w1a = w1a_ref[...].astype(bf16)
    w2a = w2a_ref[...].astype(bf16)
    wih = wih_ref[...].astype(bf16)
    whha = whha_ref[...].astype(bf16)
    ones_chunk = jnp.ones((GX_CHUNK * BB, 1), dtype=bf16)
    ones_h = jnp.ones((HB, 1), dtype=bf16)

    def sig(x):
        return 0.5 * jnp.tanh(0.5 * x) + 0.5

    def act(gates, c):
        i_t = sig(gates[:, 0 * HID:1 * HID])
        f_t = sig(gates[:, 1 * HID:2 * HID])
        g_t = jnp.tanh(gates[:, 2 * HID:3 * HID])
        o_t = sig(gates[:, 3 * HID:4 * HID])
        c_new = f_t * c + i_t * g_t
        h_new = o_t * jnp.tanh(c_new)
        return h_new, c_new

    zero = jnp.zeros((HB, HID), dtype=f32)
    hs = [zero, zero]
    cs = [zero, zero]
    for seg in range(T // T_SEG):
        for c in range(T_SEG // GX_CHUNK):
            ct = seg * T_SEG + c * GX_CHUNK
            xc = x_ref[ct:ct + GX_CHUNK].reshape(GX_CHUNK * BB, D_INA)
            z1 = jnp.maximum(jnp.dot(xc, w1a, preferred_element_type=f32).astype(bf16), 0.0)
            z1 = jnp.concatenate([z1, ones_chunk], axis=1)
            z2 = jnp.maximum(jnp.dot(z1, w2a, preferred_element_type=f32).astype(bf16), 0.0)
            gx = jnp.dot(z2, wih, preferred_element_type=f32)
            gx_scr[c * GX_CHUNK:(c + 1) * GX_CHUNK] = gx.reshape(GX_CHUNK, BB, 4 * HID)
        for t in range(T_SEG):
            gx = gx_scr[t]
            for q in range(2):
                hq = jnp.concatenate([hs[q].astype(bf16), ones_h], axis=1)
                gq = gx[q * HB:(q + 1) * HB] + jnp.dot(
                    hq, whha, preferred_element_type=f32)
                hs[q], cs[q] = act(gq, cs[q])
    h = jnp.concatenate(hs, axis=0)

    hc = jnp.maximum(jnp.dot(h, cw1_ref[...], preferred_element_type=f32)
                     + cb1_ref[...], 0.0)
    coords_ref[...] = jnp.dot(hc, cw2_ref[...], preferred_element_type=f32) + cb2_ref[...]
    hl = jnp.maximum(jnp.dot(h, lw1_ref[...], preferred_element_type=f32)
                     + lb1_ref[...], 0.0)
    labels_ref[...] = jnp.dot(hl, lw2_ref[...], preferred_element_type=f32) + lb2_ref[...]


def _full(shape):
    return pl.BlockSpec(shape, lambda i: (0,) * len(shape))


@functools.partial(jax.jit, static_argnames=("interpret",))
def _run(x, W1a, W2a, W_ih, Whha,
         coord_W1, coord_b1, coord_W2, coord_b2,
         lab_W1, lab_b1, lab_W2, lab_b2, interpret=False):
    n_tiles = B // BB
    out_shapes = (
        jax.ShapeDtypeStruct((B, 3 * STEPS), jnp.float32),
        jax.ShapeDtypeStruct((B, STEPS), jnp.float32),
    )
    return pl.pallas_call(
        _fused_kernel,
        grid=(n_tiles,),
        in_specs=[
            pl.BlockSpec((T, BB, D_INA), lambda i: (0, i, 0)),
            _full((D_INA, ENC)),
            _full((ENC + 1, ENC)),
            _full((ENC, 4 * HID)),
            _full((HID + 1, 4 * HID)),
            _full((HID, HID)), _full((1, HID)),
            _full((HID, 3 * STEPS)), _full((1, 3 * STEPS)),
            _full((HID, HID // 2)), _full((1, HID // 2)),
            _full((HID // 2, STEPS)), _full((1, STEPS)),
        ],
        out_specs=(
            pl.BlockSpec((BB, 3 * STEPS), lambda i: (i, 0)),
            pl.BlockSpec((BB, STEPS), lambda i: (i, 0)),
        ),
        out_shape=out_shapes,
        scratch_shapes=[pltpu.VMEM((T_SEG, BB, 4 * HID), jnp.float32)],
        compiler_params=pltpu.CompilerParams(
            dimension_semantics=("parallel",),
        ),
        interpret=interpret,
    )(x, W1a, W2a, W_ih, Whha,
      coord_W1, coord_b1, coord_W2, coord_b2,
      lab_W1, lab_b1, lab_W2, lab_b2)


def kernel(obs_l, obs_c, obs_m, enc_W1, enc_b1, enc_W2, enc_b2,
           W_ih, W_hh, b_ih, b_hh,
           coord_W1, coord_b1, coord_W2, coord_b2,
           lab_W1, lab_b1, lab_W2, lab_b2):
    ones = jnp.ones((B, T, 1), dtype=obs_l.dtype)
    x = jnp.concatenate([obs_l, obs_c, obs_m, ones], axis=-1)
    x = jnp.swapaxes(x, 0, 1).astype(jnp.bfloat16)
    W1a = jnp.concatenate([enc_W1, enc_b1[None, :]], axis=0)
    W2a = jnp.concatenate([enc_W2, enc_b2[None, :]], axis=0)
    Whha = jnp.concatenate([W_hh, (b_ih + b_hh)[None, :]], axis=0)
    return _run(x, W1a, W2a, W_ih, Whha,
                coord_W1, coord_b1.reshape(1, HID), coord_W2, coord_b2.reshape(1, 3 * STEPS),
                lab_W1, lab_b1.reshape(1, HID // 2), lab_W2, lab_b2.reshape(1, STEPS))


# speedup vs baseline: 1.2429x; 1.0299x over previous
import functools

import jax
import jax.numpy as jnp
from jax.experimental import pallas as pl
from jax.experimental.pallas import tpu as pltpu

B, T = 4096, 50
D_IN, ENC, HID = 11, 128, 128
STEPS = 50
BB = 512
HB = BB // 2
GX_CHUNK = 5
T_SEG = 25
D_INA = D_IN + 1


def _fused_kernel(x_ref, w1a_ref, w2a_ref, wih_ref, whha_ref,
                  cw1_ref, cb1_ref, cw2_ref, cb2_ref,
                  lw1_ref, lb1_ref, lw2_ref, lb2_ref,
                  coords_ref, labels_ref, gx_scr):
    f32 = jnp.float32
    bf16 = jnp.bfloat16
    w1a = w1a_ref[...].astype(bf16)
    w2a = w2a_ref[...].astype(bf16)
    wih = wih_ref[...].astype(bf16)
    whha = whha_ref[...].astype(bf16)
    ones_chunk = jnp.ones((GX_CHUNK * BB, 1), dtype=bf16)
    ones_h = jnp.ones((HB, 1), dtype=bf16)

    def act(gates, c):
        i_t = 0.5 * jnp.tanh(gates[:, 0 * HID:1 * HID]) + 0.5
        f_t = 0.5 * jnp.tanh(gates[:, 1 * HID:2 * HID]) + 0.5
        g_t = jnp.tanh(gates[:, 2 * HID:3 * HID])
        o_t = 0.5 * jnp.tanh(gates[:, 3 * HID:4 * HID]) + 0.5
        c_new = f_t * c + i_t * g_t
        h_new = o_t * jnp.tanh(c_new)
        return h_new, c_new

    zero = jnp.zeros((HB, HID), dtype=f32)
    hs = [zero, zero]
    cs = [zero, zero]
    for seg in range(T // T_SEG):
        for c in range(T_SEG // GX_CHUNK):
            ct = seg * T_SEG + c * GX_CHUNK
            xc = x_ref[ct:ct + GX_CHUNK].reshape(GX_CHUNK * BB, D_INA)
            z1 = jnp.maximum(jnp.dot(xc, w1a, preferred_element_type=f32).astype(bf16), 0.0)
            z1 = jnp.concatenate([z1, ones_chunk], axis=1)
            z2 = jnp.maximum(jnp.dot(z1, w2a, preferred_element_type=f32).astype(bf16), 0.0)
            gx = jnp.dot(z2, wih, preferred_element_type=f32)
            gx_scr[c * GX_CHUNK:(c + 1) * GX_CHUNK] = gx.reshape(GX_CHUNK, BB, 4 * HID)
        for t in range(T_SEG):
            gx = gx_scr[t]
            for q in range(2):
                hq = jnp.concatenate([hs[q].astype(bf16), ones_h], axis=1)
                gq = gx[q * HB:(q + 1) * HB] + jnp.dot(
                    hq, whha, preferred_element_type=f32)
                hs[q], cs[q] = act(gq, cs[q])
    h = jnp.concatenate(hs, axis=0)

    hc = jnp.maximum(jnp.dot(h, cw1_ref[...], preferred_element_type=f32)
                     + cb1_ref[...], 0.0)
    coords_ref[...] = jnp.dot(hc, cw2_ref[...], preferred_element_type=f32) + cb2_ref[...]
    hl = jnp.maximum(jnp.dot(h, lw1_ref[...], preferred_element_type=f32)
                     + lb1_ref[...], 0.0)
    labels_ref[...] = jnp.dot(hl, lw2_ref[...], preferred_element_type=f32) + lb2_ref[...]


def _full(shape):
    return pl.BlockSpec(shape, lambda i: (0,) * len(shape))


@functools.partial(jax.jit, static_argnames=("interpret",))
def _run(x, W1a, W2a, W_ih, Whha,
         coord_W1, coord_b1, coord_W2, coord_b2,
         lab_W1, lab_b1, lab_W2, lab_b2, interpret=False):
    n_tiles = B // BB
    out_shapes = (
        jax.ShapeDtypeStruct((B, 3 * STEPS), jnp.float32),
        jax.ShapeDtypeStruct((B, STEPS), jnp.float32),
    )
    return pl.pallas_call(
        _fused_kernel,
        grid=(n_tiles,),
        in_specs=[
            pl.BlockSpec((T, BB, D_INA), lambda i: (0, i, 0)),
            _full((D_INA, ENC)),
            _full((ENC + 1, ENC)),
            _full((ENC, 4 * HID)),
            _full((HID + 1, 4 * HID)),
            _full((HID, HID)), _full((1, HID)),
            _full((HID, 3 * STEPS)), _full((1, 3 * STEPS)),
            _full((HID, HID // 2)), _full((1, HID // 2)),
            _full((HID // 2, STEPS)), _full((1, STEPS)),
        ],
        out_specs=(
            pl.BlockSpec((BB, 3 * STEPS), lambda i: (i, 0)),
            pl.BlockSpec((BB, STEPS), lambda i: (i, 0)),
        ),
        out_shape=out_shapes,
        scratch_shapes=[pltpu.VMEM((T_SEG, BB, 4 * HID), jnp.float32)],
        compiler_params=pltpu.CompilerParams(
            dimension_semantics=("parallel",),
        ),
        interpret=interpret,
    )(x, W1a, W2a, W_ih, Whha,
      coord_W1, coord_b1, coord_W2, coord_b2,
      lab_W1, lab_b1, lab_W2, lab_b2)


def kernel(obs_l, obs_c, obs_m, enc_W1, enc_b1, enc_W2, enc_b2,
           W_ih, W_hh, b_ih, b_hh,
           coord_W1, coord_b1, coord_W2, coord_b2,
           lab_W1, lab_b1, lab_W2, lab_b2):
    ones = jnp.ones((B, T, 1), dtype=obs_l.dtype)
    x = jnp.concatenate([obs_l, obs_c, obs_m, ones], axis=-1)
    x = jnp.swapaxes(x, 0, 1).astype(jnp.bfloat16)
    W1a = jnp.concatenate([enc_W1, enc_b1[None, :]], axis=0)
    W2a = jnp.concatenate([enc_W2, enc_b2[None, :]], axis=0)
    gate_scale = jnp.concatenate([
        jnp.full((HID,), 0.5, jnp.float32),
        jnp.full((HID,), 0.5, jnp.float32),
        jnp.ones((HID,), jnp.float32),
        jnp.full((HID,), 0.5, jnp.float32)])
    Whha = jnp.concatenate([W_hh, (b_ih + b_hh)[None, :]], axis=0) * gate_scale[None, :]
    return _run(x, W1a, W2a, W_ih * gate_scale[None, :], Whha,
                coord_W1, coord_b1.reshape(1, HID), coord_W2, coord_b2.reshape(1, 3 * STEPS),
                lab_W1, lab_b1.reshape(1, HID // 2), lab_W2, lab_b2.reshape(1, STEPS))
